# Initial kernel scaffold; baseline (speedup 1.0000x reference)
import jax
import jax.numpy as jnp
from jax.experimental import pallas as pl


def kernel(inputs, indices, output_shape):
    raise NotImplementedError("write your pallas kernel here")


# speedup vs baseline: 17.2322x; 17.2322x over previous
import jax
import jax.numpy as jnp
from jax import lax
from jax.experimental import pallas as pl
from jax.experimental.pallas import tpu as pltpu
from jax.experimental.pallas import tpu_sc as plsc

_B = 8
_N = 112 * 112 * 96
_R = 224 * 224 * 96
_Q = _R // 4
_PT = _N // 16
_WSZ = 9408
_WN = _PT // _WSZ
_PAD = 1024


def _sc_body(val_hbm, idx_hbm, out_hbm, acc, idxbuf, valbuf, zbuf):
    c = lax.axis_index("c")
    s = lax.axis_index("s")

    def zb(i, carry):
        zbuf[pl.ds(i * 16, 16)] = jnp.zeros((16,), jnp.float32)
        return carry

    lax.fori_loop(0, _WSZ // 16, zb, 0)

    iota = lax.iota(jnp.int32, 16)

    def per_bp(b, p):
        chunk = 2 * p + c
        cbase = chunk * _Q
        out_base = b * _R + chunk * _Q
        def zwin(k, carry):
            pltpu.sync_copy(zbuf, acc.at[pl.ds(s * _PT + k * _WSZ, _WSZ)])
            return carry

        lax.fori_loop(0, _WN, zwin, 0)
        plsc.subcore_barrier()

        def win(w, carry):
            off = b * _N + s * _PT + w * _WSZ
            pltpu.sync_copy(idx_hbm.at[pl.ds(off, _WSZ)], idxbuf)
            pltpu.sync_copy(val_hbm.at[pl.ds(off, _WSZ)], valbuf)

            def xform(j, carry2):
                iv = idxbuf[pl.ds(j * 16, 16)]
                lv = iv - cbase
                inr = (lv >= 0) & (lv < _Q)
                pad = _Q + ((j * 16) % _PAD) + iota
                idxbuf[pl.ds(j * 16, 16)] = jnp.where(inr, lv, pad)
                return carry2

            lax.fori_loop(0, _WSZ // 16, xform, 0)
            pltpu.sync_copy(valbuf, acc.at[idxbuf], add=True)
            return carry

        lax.fori_loop(0, _WN, win, 0)
        plsc.subcore_barrier()

        def owin(k, carry):
            pltpu.sync_copy(acc.at[pl.ds(s * _PT + k * _WSZ, _WSZ)], valbuf)
            pltpu.sync_copy(
                valbuf, out_hbm.at[pl.ds(out_base + s * _PT + k * _WSZ, _WSZ)])
            return carry

        lax.fori_loop(0, _WN, owin, 0)
        plsc.subcore_barrier()

    def per_b(b, carry):
        per_bp(b, 0)
        per_bp(b, 1)
        return carry

    lax.fori_loop(0, _B, per_b, 0)


def kernel(inputs, indices, output_shape):
    del output_shape
    batch, _, _, chan = inputs.shape
    val = inputs.reshape(-1)
    idx = indices.reshape(-1).astype(jnp.int32)
    mesh = plsc.VectorSubcoreMesh(core_axis_name="c", subcore_axis_name="s")
    out = pl.kernel(
        _sc_body,
        out_type=jax.ShapeDtypeStruct((_B * _R,), jnp.float32),
        mesh=mesh,
        scratch_types=[
            pltpu.VMEM_SHARED((_Q + _PAD,), jnp.float32),
            pltpu.VMEM((_WSZ,), jnp.int32),
            pltpu.VMEM((_WSZ,), jnp.float32),
            pltpu.VMEM((_WSZ,), jnp.float32),
        ],
    )(val, idx)
    return out.reshape(batch, 224, 224, chan)


# speedup vs baseline: 22.3162x; 1.2950x over previous
import jax
import jax.numpy as jnp
from jax import lax
from jax.experimental import pallas as pl
from jax.experimental.pallas import tpu as pltpu
from jax.experimental.pallas import tpu_sc as plsc

_B = 8
_N = 112 * 112 * 96
_R = 224 * 224 * 96
_U = _R // 32
_PT = _N // 16
_WSZ = 4704
_WN = _PT // _WSZ
_OSZ = 4704
_ACC = 11 * _U
_PAD = 2048


def _sc_body(val_hbm, idx_hbm, out_hbm, acc,
             idx0, val0, idx1, val1, obuf, sem0, sem1):
    c = lax.axis_index("c")
    s = lax.axis_index("s")
    iota = lax.iota(jnp.int32, 16)
    zero16 = jnp.zeros((16,), jnp.float32)

    def zv(i, carry):
        obuf[pl.ds(i * 16, 16)] = zero16
        return carry

    lax.fori_loop(0, _OSZ // 16, zv, 0)

    def task(b, cbase, csz_u):
        csz = csz_u * _U
        out_base = b * _R + cbase
        tile_w = csz_u * (_U // 16)

        def zwin(k, carry):
            pltpu.sync_copy(obuf, acc.at[pl.ds(s * tile_w + k * _OSZ, _OSZ)])
            return carry

        lax.fori_loop(0, tile_w // _OSZ, zwin, 0)
        plsc.subcore_barrier()

        base_hbm = b * _N + s * _PT

        def fire(w, ib, vb, sem):
            pltpu.async_copy(idx_hbm.at[pl.ds(base_hbm + w * _WSZ, _WSZ)],
                             ib, sem)
            pltpu.async_copy(val_hbm.at[pl.ds(base_hbm + w * _WSZ, _WSZ)],
                             vb, sem)

        def drain(ib, vb, sem):
            pltpu.make_async_copy(idx_hbm.at[pl.ds(0, _WSZ)], ib, sem).wait()
            pltpu.make_async_copy(val_hbm.at[pl.ds(0, _WSZ)], vb, sem).wait()

        def xform(ib):
            def body(j, carry):
                lv = ib[pl.ds(j * 16, 16)] - cbase
                m = lv.astype(jnp.uint32) < csz.astype(jnp.uint32)
                padv = csz + ((j * 16) % _PAD) + iota
                ib[pl.ds(j * 16, 16)] = jnp.where(m, lv, padv)
                return carry

            lax.fori_loop(0, _WSZ // 16, body, 0)

        fire(0, idx0, val0, sem0)

        def block(g, carry):
            w0 = g * 2
            drain(idx0, val0, sem0)
            fire(w0 + 1, idx1, val1, sem1)
            xform(idx0)
            pltpu.sync_copy(val0, acc.at[idx0], add=True)
            drain(idx1, val1, sem1)

            @pl.when(g + 1 < _WN // 2)
            def _():
                fire(w0 + 2, idx0, val0, sem0)

            xform(idx1)
            pltpu.sync_copy(val1, acc.at[idx1], add=True)
            return carry

        lax.fori_loop(0, _WN // 2, block, 0)
        plsc.subcore_barrier()

        def owin(k, carry):
            pltpu.sync_copy(acc.at[pl.ds(s * tile_w + k * _OSZ, _OSZ)], obuf)
            pltpu.sync_copy(
                obuf, out_hbm.at[pl.ds(out_base + s * tile_w + k * _OSZ, _OSZ)])
            return carry

        lax.fori_loop(0, tile_w // _OSZ, owin, 0)
        plsc.subcore_barrier()

        def rez(i, carry):
            obuf[pl.ds(i * 16, 16)] = zero16
            return carry

        lax.fori_loop(0, _OSZ // 16, rez, 0)

    def per_b(b, carry):
        even = ((b + c) % 2) == 0

        def one(k, carry2):
            cbase = jnp.where(even, 0, jnp.where(k == 0, 11 * _U, 22 * _U))
            csz_u = jnp.where(even, 11, jnp.where(k == 0, 11, 10))
            task(b, cbase, csz_u)
            return carry2

        ntask = jnp.where(even, 1, 2)
        lax.fori_loop(0, ntask, one, 0)
        return carry

    lax.fori_loop(0, _B, per_b, 0)


def kernel(inputs, indices, output_shape):
    del output_shape
    batch, _, _, chan = inputs.shape
    val = inputs.reshape(-1)
    idx = indices.reshape(-1).astype(jnp.int32)
    mesh = plsc.VectorSubcoreMesh(core_axis_name="c", subcore_axis_name="s")
    out = pl.kernel(
        _sc_body,
        out_type=jax.ShapeDtypeStruct((_B * _R,), jnp.float32),
        mesh=mesh,
        scratch_types=[
            pltpu.VMEM_SHARED((_ACC + _PAD,), jnp.float32),
            pltpu.VMEM((_WSZ,), jnp.int32),
            pltpu.VMEM((_WSZ,), jnp.float32),
            pltpu.VMEM((_WSZ,), jnp.int32),
            pltpu.VMEM((_WSZ,), jnp.float32),
            pltpu.VMEM((_OSZ,), jnp.float32),
            pltpu.SemaphoreType.DMA,
            pltpu.SemaphoreType.DMA,
        ],
    )(val, idx)
    return out.reshape(batch, 224, 224, chan)
